# baseline (speedup 1.0000x reference)
import functools

import jax
import jax.numpy as jnp
from jax import lax
from jax.experimental import pallas as pl
from jax.experimental.pallas import tpu as pltpu
from jax.experimental.pallas import tpu_sc as plsc

N = 10000
D_IN, D_HID, D_OUT = 128, 64, 16
E = 320000
NC, NS, L = 2, 16, 16
NW = NC * NS
CH = 128
CPT = 80
EPAD = NW * CPT * CH
PADV = N
NROWS = 10240
RPS = NROWS // NS
NSTG = 10016

_mesh = plsc.VectorSubcoreMesh(core_axis_name="c", subcore_axis_name="s")
_f32 = jnp.float32
_sc_params = pltpu.CompilerParams(use_tc_tiling_on_sc=False)


@functools.partial(
    pl.kernel,
    out_type=jax.ShapeDtypeStruct((NC, NROWS, L), _f32),
    mesh=_mesh,
    scratch_types=[
        pltpu.VMEM((CPT, CH), jnp.int32),
        pltpu.VMEM((CH, L), _f32),
        pltpu.VMEM((CH, L), _f32),
        pltpu.VMEM_SHARED((NROWS, L), _f32),
        pltpu.SemaphoreType.DMA,
    ],
    compiler_params=_sc_params,
)
def _sc_hist(edge_hbm, out_hbm, dstv, ones, zbuf, hist_sh, sem):
    cid = lax.axis_index("c")
    sid = lax.axis_index("s")
    wid = cid * NS + sid

    @pl.loop(0, CH)
    def _(i):
        ones.at[i][...] = jnp.full((L,), 1.0, _f32)
        zbuf.at[i][...] = jnp.zeros((L,), _f32)

    @pl.loop(0, RPS // CH)
    def _(k):
        pltpu.sync_copy(zbuf, hist_sh.at[pl.ds(sid * RPS + k * CH, CH)])

    pltpu.sync_copy(edge_hbm.at[1, wid], dstv)
    plsc.subcore_barrier()

    NF = 16

    def fire(base):
        @pl.loop(0, NF)
        def _(i):
            pltpu.async_copy(ones, hist_sh.at[dstv.at[base + i]], sem,
                             add=True)

    def drain():
        @pl.loop(0, NF)
        def _(i):
            pltpu.make_async_copy(ones, hist_sh.at[dstv.at[0]], sem).wait()

    fire(0)

    @pl.loop(0, CPT - NF, step=NF)
    def _(j):
        fire(j + NF)
        drain()

    drain()
    plsc.subcore_barrier()
    pltpu.sync_copy(hist_sh.at[pl.ds(sid * RPS, RPS)],
                    out_hbm.at[cid, pl.ds(sid * RPS, RPS)])


def _make_sc_agg(d_feat, grp):
    assert CPT % (2 * grp) == 0
    n_bufs = 2 * grp
    scratch = (
        [pltpu.VMEM((CPT, CH), jnp.int32)]
        + [pltpu.VMEM((CH, d_feat), _f32) for _ in range(n_bufs)]
        + [pltpu.VMEM((grp, CH), jnp.int32),
           pltpu.VMEM((grp, CH), jnp.int32),
           pltpu.VMEM_SHARED((NROWS, d_feat), _f32),
           pltpu.VMEM_SHARED((NSTG, d_feat), _f32),
           pltpu.SemaphoreType.DMA,
           pltpu.SemaphoreType.DMA,
           pltpu.SemaphoreType.DMA,
           pltpu.SemaphoreType.DMA,
           pltpu.SemaphoreType.DMA,
           pltpu.SemaphoreType.DMA]
    )

    @functools.partial(
        pl.kernel,
        out_type=jax.ShapeDtypeStruct((NC, NROWS, 128), _f32),
        mesh=_mesh,
        scratch_types=scratch,
        compiler_params=_sc_params,
    )
    def _sc_agg(g_hbm, edge_hbm, out_hbm, srcv, *rest):
        buf_a = rest[:grp]
        buf_b = rest[grp:n_bufs]
        (dvb_a, dvb_b, acc_sh, g_sh, sem_ga, sem_gb, sem_sa, sem_sb,
         sem_ia, sem_ib) = rest[n_bufs:n_bufs + 10]
        sem_i = {id(dvb_a): sem_ia, id(dvb_b): sem_ib}
        zbuf = buf_a[0]

        cid = lax.axis_index("c")
        sid = lax.axis_index("s")
        wid = cid * NS + sid

        def fire_gathers(bufs, dvb, sem, base):
            for i in range(grp):
                pltpu.async_copy(g_sh.at[srcv.at[base + i]], bufs[i], sem)
            pltpu.async_copy(edge_hbm.at[1, wid, pl.ds(base, grp)], dvb,
                             sem_i[id(dvb)])

        def drain_gathers(bufs, dvb, sem):
            for i in range(grp):
                pltpu.make_async_copy(g_sh.at[srcv.at[0]], bufs[i],
                                      sem).wait()
            pltpu.make_async_copy(edge_hbm.at[1, wid, pl.ds(0, grp)], dvb,
                                  sem_i[id(dvb)]).wait()

        def fire_scatters(bufs, dvb, sem):
            for i in range(grp):
                pltpu.async_copy(bufs[i], acc_sh.at[dvb.at[i]], sem,
                                 add=True)

        def drain_scatters(bufs, dvb, sem):
            for i in range(grp):
                pltpu.make_async_copy(bufs[i], acc_sh.at[dvb.at[0]],
                                      sem).wait()

        @pl.loop(0, CH)
        def _(i):
            @pl.loop(0, d_feat, step=L)
            def _(c):
                zbuf.at[i, pl.ds(c, L)][...] = jnp.zeros((L,), _f32)

        @pl.loop(0, RPS // CH)
        def _(k):
            pltpu.sync_copy(zbuf, acc_sh.at[pl.ds(sid * RPS + k * CH, CH)])

        pltpu.sync_copy(edge_hbm.at[0, wid], srcv)
        rps_g = N // NS
        pltpu.sync_copy(
            g_hbm.at[pl.ds(sid * rps_g, rps_g), pl.ds(0, d_feat)],
            g_sh.at[pl.ds(sid * rps_g, rps_g)])
        plsc.subcore_barrier()

        fire_gathers(buf_a, dvb_a, sem_ga, 0)

        @pl.loop(0, CPT, step=2 * grp)
        def _(j):
            drain_gathers(buf_a, dvb_a, sem_ga)
            fire_scatters(buf_a, dvb_a, sem_sa)

            @pl.when(j > 0)
            def _():
                drain_scatters(buf_b, dvb_b, sem_sb)

            fire_gathers(buf_b, dvb_b, sem_gb, j + grp)
            drain_gathers(buf_b, dvb_b, sem_gb)
            fire_scatters(buf_b, dvb_b, sem_sb)
            drain_scatters(buf_a, dvb_a, sem_sa)

            @pl.when(j + 2 * grp < CPT)
            def _():
                fire_gathers(buf_a, dvb_a, sem_ga, j + 2 * grp)

        drain_scatters(buf_b, dvb_b, sem_sb)
        plsc.subcore_barrier()
        pltpu.sync_copy(acc_sh.at[pl.ds(sid * RPS, RPS)],
                        out_hbm.at[cid, pl.ds(sid * RPS, RPS),
                                   pl.ds(0, d_feat)])

    return _sc_agg


_sc_agg1 = _make_sc_agg(D_HID, 2)
_sc_agg2 = _make_sc_agg(D_OUT, 8)


def _mm1_body(x_ref, w_ref, o_ref):
    o_ref[...] = jnp.dot(x_ref[...], w_ref[...],
                         preferred_element_type=_f32)


NB = 10
BR = N // NB


def _dis(hist_ref):
    deg = hist_ref[0, :, 0:1] + hist_ref[1, :, 0:1] + 1.0
    return lax.rsqrt(deg)


def _scale_body(hist_ref, h_ref, g_ref):
    g_ref[:, 0:D_HID] = _dis(hist_ref) * h_ref[...]


def _mid_body(hist_ref, acc_ref, h1_ref, b1_ref, w2_ref, h2_ref, g2_ref):
    dis = _dis(hist_ref)
    s = acc_ref[0, :, 0:D_HID] + acc_ref[1, :, 0:D_HID]
    out1 = jnp.maximum(dis * s + (dis * dis) * h1_ref[...] + b1_ref[...], 0.0)
    h2 = jnp.dot(out1, w2_ref[...], preferred_element_type=_f32)
    h2_ref[...] = h2
    g2_ref[:, 0:D_OUT] = dis * h2


def _out_body(hist_ref, acc_ref, h2_ref, b2_ref, o_ref):
    dis = _dis(hist_ref)
    s = acc_ref[0, :, 0:D_OUT] + acc_ref[1, :, 0:D_OUT]
    o_ref[...] = dis * s + (dis * dis) * h2_ref[...] + b2_ref[...]


def _hist_spec():
    return pl.BlockSpec((NC, BR, L), lambda i: (0, i, 0))


def _acc_spec():
    return pl.BlockSpec((NC, BR, 128), lambda i: (0, i, 0))


def _row_spec(d):
    return pl.BlockSpec((BR, d), lambda i: (i, 0))


def _full_spec(a, b):
    return pl.BlockSpec((a, b), lambda i: (0, 0))


def kernel(x, edge_index, W1, b1, W2, b2):
    ei = jnp.pad(edge_index.astype(jnp.int32), ((0, 0), (0, EPAD - E)),
                 constant_values=PADV)
    edge4 = ei.reshape(2, NW, CPT, CH)

    hist = _sc_hist(edge4)

    h1 = pl.pallas_call(
        _mm1_body,
        out_shape=jax.ShapeDtypeStruct((N, D_HID), _f32),
    )(x, W1)

    g1 = pl.pallas_call(
        _scale_body,
        grid=(NB,),
        in_specs=[_hist_spec(), _row_spec(D_HID)],
        out_specs=_row_spec(128),
        out_shape=jax.ShapeDtypeStruct((N, 128), _f32),
    )(hist, h1)

    acc1 = _sc_agg1(g1, edge4)

    h2, g2 = pl.pallas_call(
        _mid_body,
        grid=(NB,),
        in_specs=[_hist_spec(), _acc_spec(), _row_spec(D_HID),
                  _full_spec(1, D_HID), _full_spec(D_HID, D_OUT)],
        out_specs=[_row_spec(D_OUT), _row_spec(128)],
        out_shape=[jax.ShapeDtypeStruct((N, D_OUT), _f32),
                   jax.ShapeDtypeStruct((N, 128), _f32)],
    )(hist, acc1, h1, b1.reshape(1, D_HID), W2)

    acc2 = _sc_agg2(g2, edge4)

    out = pl.pallas_call(
        _out_body,
        grid=(NB,),
        in_specs=[_hist_spec(), _acc_spec(), _row_spec(D_OUT),
                  _full_spec(1, D_OUT)],
        out_specs=_row_spec(D_OUT),
        out_shape=jax.ShapeDtypeStruct((N, D_OUT), _f32),
    )(hist, acc2, h2, b2.reshape(1, D_OUT))

    return out


# speedup vs baseline: 1.0127x; 1.0127x over previous
import functools

import jax
import jax.numpy as jnp
from jax import lax
from jax.experimental import pallas as pl
from jax.experimental.pallas import tpu as pltpu
from jax.experimental.pallas import tpu_sc as plsc

N = 10000
D_IN, D_HID, D_OUT = 128, 64, 16
E = 320000
NC, NS, L = 2, 16, 16
NW = NC * NS
CH = 128
CPT = 80
EPAD = NW * CPT * CH
PADV = N
NROWS = 10240
RPS = NROWS // NS
NSTG = 10016

_mesh = plsc.VectorSubcoreMesh(core_axis_name="c", subcore_axis_name="s")
_f32 = jnp.float32
_sc_params = pltpu.CompilerParams(use_tc_tiling_on_sc=False)


@functools.partial(
    pl.kernel,
    out_type=jax.ShapeDtypeStruct((NC, NROWS, L), _f32),
    mesh=_mesh,
    scratch_types=[
        pltpu.VMEM((CPT, CH), jnp.int32),
        pltpu.VMEM((CH, L), _f32),
        pltpu.VMEM((CH, L), _f32),
        pltpu.VMEM_SHARED((NROWS, L), _f32),
        pltpu.SemaphoreType.DMA,
    ],
    compiler_params=_sc_params,
)
def _sc_hist(edge_hbm, out_hbm, dstv, ones, zbuf, hist_sh, sem):
    cid = lax.axis_index("c")
    sid = lax.axis_index("s")
    wid = cid * NS + sid

    @pl.loop(0, CH)
    def _(i):
        ones.at[i][...] = jnp.full((L,), 1.0, _f32)
        zbuf.at[i][...] = jnp.zeros((L,), _f32)

    @pl.loop(0, RPS // CH)
    def _(k):
        pltpu.sync_copy(zbuf, hist_sh.at[pl.ds(sid * RPS + k * CH, CH)])

    pltpu.sync_copy(edge_hbm.at[1, wid], dstv)
    plsc.subcore_barrier()

    NF = 16

    def fire(base):
        @pl.loop(0, NF)
        def _(i):
            pltpu.async_copy(ones, hist_sh.at[dstv.at[base + i]], sem,
                             add=True)

    def drain():
        @pl.loop(0, NF)
        def _(i):
            pltpu.make_async_copy(ones, hist_sh.at[dstv.at[0]], sem).wait()

    fire(0)

    @pl.loop(0, CPT - NF, step=NF)
    def _(j):
        fire(j + NF)
        drain()

    drain()
    plsc.subcore_barrier()
    pltpu.sync_copy(hist_sh.at[pl.ds(sid * RPS, RPS)],
                    out_hbm.at[cid, pl.ds(sid * RPS, RPS)])


def _make_sc_agg(d_feat, grp):
    assert CPT % (2 * grp) == 0
    n_bufs = 2 * grp
    scratch = (
        [pltpu.VMEM((CPT, CH), jnp.int32)]
        + [pltpu.VMEM((CH, d_feat), _f32) for _ in range(n_bufs)]
        + [pltpu.VMEM((grp, CH), jnp.int32),
           pltpu.VMEM((grp, CH), jnp.int32),
           pltpu.VMEM_SHARED((NROWS, d_feat), _f32),
           pltpu.VMEM_SHARED((NSTG, d_feat), _f32),
           pltpu.SemaphoreType.DMA,
           pltpu.SemaphoreType.DMA,
           pltpu.SemaphoreType.DMA,
           pltpu.SemaphoreType.DMA,
           pltpu.SemaphoreType.DMA,
           pltpu.SemaphoreType.DMA]
    )

    @functools.partial(
        pl.kernel,
        out_type=jax.ShapeDtypeStruct((NC, NROWS, 128), _f32),
        mesh=_mesh,
        scratch_types=scratch,
        compiler_params=_sc_params,
    )
    def _sc_agg(g_hbm, edge_hbm, out_hbm, srcv, *rest):
        buf_a = rest[:grp]
        buf_b = rest[grp:n_bufs]
        (dvb_a, dvb_b, acc_sh, g_sh, sem_ga, sem_gb, sem_sa, sem_sb,
         sem_ia, sem_ib) = rest[n_bufs:n_bufs + 10]
        sem_i = {id(dvb_a): sem_ia, id(dvb_b): sem_ib}
        zbuf = buf_a[0]

        cid = lax.axis_index("c")
        sid = lax.axis_index("s")
        wid = cid * NS + sid

        def fire_gathers(bufs, dvb, sem, base):
            for i in range(grp):
                pltpu.async_copy(g_sh.at[srcv.at[base + i]], bufs[i], sem)
            pltpu.async_copy(edge_hbm.at[1, wid, pl.ds(base, grp)], dvb,
                             sem_i[id(dvb)])

        def drain_gathers(bufs, dvb, sem):
            for i in range(grp):
                pltpu.make_async_copy(g_sh.at[srcv.at[0]], bufs[i],
                                      sem).wait()
            pltpu.make_async_copy(edge_hbm.at[1, wid, pl.ds(0, grp)], dvb,
                                  sem_i[id(dvb)]).wait()

        def fire_scatters(bufs, dvb, sem):
            for i in range(grp):
                pltpu.async_copy(bufs[i], acc_sh.at[dvb.at[i]], sem,
                                 add=True)

        def drain_scatters(bufs, dvb, sem):
            for i in range(grp):
                pltpu.make_async_copy(bufs[i], acc_sh.at[dvb.at[0]],
                                      sem).wait()

        @pl.loop(0, CH)
        def _(i):
            @pl.loop(0, d_feat, step=L)
            def _(c):
                zbuf.at[i, pl.ds(c, L)][...] = jnp.zeros((L,), _f32)

        @pl.loop(0, RPS // CH)
        def _(k):
            pltpu.sync_copy(zbuf, acc_sh.at[pl.ds(sid * RPS + k * CH, CH)])

        pltpu.sync_copy(edge_hbm.at[0, wid], srcv)
        rps_g = N // NS
        pltpu.sync_copy(
            g_hbm.at[pl.ds(sid * rps_g, rps_g), pl.ds(0, d_feat)],
            g_sh.at[pl.ds(sid * rps_g, rps_g)])
        plsc.subcore_barrier()

        fire_gathers(buf_a, dvb_a, sem_ga, 0)

        @pl.loop(0, CPT, step=2 * grp)
        def _(j):
            drain_gathers(buf_a, dvb_a, sem_ga)
            fire_scatters(buf_a, dvb_a, sem_sa)

            @pl.when(j > 0)
            def _():
                drain_scatters(buf_b, dvb_b, sem_sb)

            fire_gathers(buf_b, dvb_b, sem_gb, j + grp)
            drain_gathers(buf_b, dvb_b, sem_gb)
            fire_scatters(buf_b, dvb_b, sem_sb)
            drain_scatters(buf_a, dvb_a, sem_sa)

            @pl.when(j + 2 * grp < CPT)
            def _():
                fire_gathers(buf_a, dvb_a, sem_ga, j + 2 * grp)

        drain_scatters(buf_b, dvb_b, sem_sb)
        plsc.subcore_barrier()
        pltpu.sync_copy(acc_sh.at[pl.ds(sid * RPS, RPS)],
                        out_hbm.at[cid, pl.ds(sid * RPS, RPS),
                                   pl.ds(0, d_feat)])

    return _sc_agg


_sc_agg1 = _make_sc_agg(D_HID, 2)
_sc_agg2 = _make_sc_agg(D_OUT, 8)


def _mm1_body(x_ref, w_ref, o_ref):
    o_ref[...] = jnp.dot(x_ref[...], w_ref[...],
                         preferred_element_type=_f32)


def _dis(hist_ref):
    deg = hist_ref[0, 0:N, 0:1] + hist_ref[1, 0:N, 0:1] + 1.0
    return lax.rsqrt(deg)


def _scale_body(hist_ref, h_ref, g_ref):
    g_ref[:, 0:D_HID] = _dis(hist_ref) * h_ref[...]


def _mid_body(hist_ref, acc_ref, h1_ref, b1_ref, w2_ref, h2_ref, g2_ref):
    dis = _dis(hist_ref)
    s = acc_ref[0, 0:N, 0:D_HID] + acc_ref[1, 0:N, 0:D_HID]
    out1 = jnp.maximum(dis * s + (dis * dis) * h1_ref[...] + b1_ref[...], 0.0)
    h2 = jnp.dot(out1, w2_ref[...], preferred_element_type=_f32)
    h2_ref[...] = h2
    g2_ref[:, 0:D_OUT] = dis * h2


def _out_body(hist_ref, acc_ref, h2_ref, b2_ref, o_ref):
    dis = _dis(hist_ref)
    s = acc_ref[0, 0:N, 0:D_OUT] + acc_ref[1, 0:N, 0:D_OUT]
    o_ref[...] = dis * s + (dis * dis) * h2_ref[...] + b2_ref[...]


def kernel(x, edge_index, W1, b1, W2, b2):
    ei = jnp.pad(edge_index.astype(jnp.int32), ((0, 0), (0, EPAD - E)),
                 constant_values=PADV)
    edge4 = ei.reshape(2, NW, CPT, CH)

    hist = _sc_hist(edge4)

    h1 = pl.pallas_call(
        _mm1_body,
        out_shape=jax.ShapeDtypeStruct((N, D_HID), _f32),
    )(x, W1)

    g1 = pl.pallas_call(
        _scale_body,
        out_shape=jax.ShapeDtypeStruct((N, 128), _f32),
    )(hist, h1)

    acc1 = _sc_agg1(g1, edge4)

    h2, g2 = pl.pallas_call(
        _mid_body,
        out_shape=[jax.ShapeDtypeStruct((N, D_OUT), _f32),
                   jax.ShapeDtypeStruct((N, 128), _f32)],
    )(hist, acc1, h1, b1.reshape(1, D_HID), W2)

    acc2 = _sc_agg2(g2, edge4)

    out = pl.pallas_call(
        _out_body,
        out_shape=jax.ShapeDtypeStruct((N, D_OUT), _f32),
    )(hist, acc2, h2, b2.reshape(1, D_OUT))

    return out
